# baseline (speedup 1.0000x reference)
import functools

import jax
import jax.numpy as jnp
from jax import lax
from jax.experimental import pallas as pl
from jax.experimental.pallas import tpu as pltpu
from jax.experimental.pallas import tpu_sc as plsc

_BLOCK = 8192


def _score_body(vis_ref, db_ref, out_ref, best_val, best_idx, *, block, n_db):
    i = pl.program_id(0)

    @pl.when(i == 0)
    def _init():
        best_val[...] = jnp.full(best_val.shape, -jnp.inf, jnp.float32)
        best_idx[...] = jnp.zeros(best_idx.shape, jnp.int32)

    db = db_ref[...]
    dots = lax.dot_general(
        vis_ref[...], db, (((1,), (1,)), ((), ())),
        preferred_element_type=jnp.float32,
    )
    norm2 = lax.dot_general(
        jnp.ones((1, db.shape[1]), jnp.float32), db * db,
        (((1,), (1,)), ((), ())),
        preferred_element_type=jnp.float32,
        precision=lax.Precision.HIGHEST,
    )
    scores = dots / jnp.sqrt(norm2)
    col = i * block + lax.broadcasted_iota(jnp.int32, scores.shape, 1)
    scores = jnp.where(col < n_db, scores, -jnp.inf)

    loc_max = jnp.max(scores, axis=1, keepdims=True)
    loc_arg = (i * block + jnp.argmax(scores, axis=1).astype(jnp.int32))[:, None]

    upd = loc_max > best_val[...]
    best_val[...] = jnp.where(upd, loc_max, best_val[...])
    best_idx[...] = jnp.where(upd, loc_arg, best_idx[...])
    out_ref[...] = best_idx[...]


def _best_index(vis, train_db, block=_BLOCK):
    q, feat = vis.shape
    n_db = train_db.shape[0]
    grid = (n_db + block - 1) // block
    return pl.pallas_call(
        functools.partial(_score_body, block=block, n_db=n_db),
        grid=(grid,),
        in_specs=[
            pl.BlockSpec((q, feat), lambda i: (0, 0)),
            pl.BlockSpec((block, feat), lambda i: (i, 0)),
        ],
        out_specs=pl.BlockSpec((q, 1), lambda i: (0, 0)),
        out_shape=jax.ShapeDtypeStruct((q, 1), jnp.int32),
        scratch_shapes=[
            pltpu.VMEM((q, 1), jnp.float32),
            pltpu.VMEM((q, 1), jnp.int32),
        ],
    )(vis, train_db)


def _gather_captions(lang_ids, best):
    n = best.shape[0]
    mesh = plsc.VectorSubcoreMesh(core_axis_name="c", subcore_axis_name="s")

    @functools.partial(
        pl.kernel,
        mesh=mesh,
        out_type=jax.ShapeDtypeStruct((n,), lang_ids.dtype),
        scratch_types=[
            pltpu.VMEM((n,), jnp.int32),
            pltpu.VMEM((n,), lang_ids.dtype),
            pltpu.SemaphoreType.DMA,
        ],
    )
    def k(lang_hbm, idx_hbm, out_hbm, idx_v, vals_v, sem):
        first = (lax.axis_index("c") == 0) & (lax.axis_index("s") == 0)

        @pl.when(first)
        def _():
            pltpu.sync_copy(idx_hbm, idx_v)
            pltpu.async_copy(lang_hbm.at[idx_v], vals_v, sem).wait()
            pltpu.sync_copy(vals_v, out_hbm)

    return k(lang_ids, best)


def kernel(t_feat, train_db, lang_ids):
    vis = t_feat[:, :-4]
    best = _best_index(vis, train_db).reshape(-1)
    return _gather_captions(lang_ids, best)


# speedup vs baseline: 1.6957x; 1.6957x over previous
import functools

import jax
import jax.numpy as jnp
from jax import lax
from jax.experimental import pallas as pl
from jax.experimental.pallas import tpu as pltpu
from jax.experimental.pallas import tpu_sc as plsc

_BLOCK = 8192


def _score_body(vis_ref, db_ref, out_ref, best_val, best_idx, *, block, n_db):
    i = pl.program_id(0)

    @pl.when(i == 0)
    def _init():
        best_val[...] = jnp.full(best_val.shape, -jnp.inf, jnp.float32)
        best_idx[...] = jnp.zeros(best_idx.shape, jnp.int32)

    db = db_ref[...]
    dots_t = lax.dot_general(
        db, vis_ref[...], (((1,), (1,)), ((), ())),
        preferred_element_type=jnp.float32,
    )
    n2_t = lax.dot_general(
        db * db, jnp.ones((8, db.shape[1]), jnp.float32),
        (((1,), (1,)), ((), ())),
        preferred_element_type=jnp.float32,
    )
    dots = dots_t.T
    norm2 = n2_t.T[0:1, :]
    scores = dots / jnp.sqrt(norm2)
    col = i * block + lax.broadcasted_iota(jnp.int32, scores.shape, 1)
    scores = jnp.where(col < n_db, scores, -jnp.inf)

    loc_max = jnp.max(scores, axis=1, keepdims=True)
    loc_arg = (i * block + jnp.argmax(scores, axis=1).astype(jnp.int32))[:, None]

    upd = loc_max > best_val[...]
    best_val[...] = jnp.where(upd, loc_max, best_val[...])
    best_idx[...] = jnp.where(upd, loc_arg, best_idx[...])
    out_ref[...] = best_idx[...]


def _best_index(vis, train_db, block=_BLOCK):
    q, feat = vis.shape
    n_db = train_db.shape[0]
    grid = (n_db + block - 1) // block
    return pl.pallas_call(
        functools.partial(_score_body, block=block, n_db=n_db),
        grid=(grid,),
        in_specs=[
            pl.BlockSpec((q, feat), lambda i: (0, 0)),
            pl.BlockSpec((block, feat), lambda i: (i, 0)),
        ],
        out_specs=pl.BlockSpec((q, 1), lambda i: (0, 0)),
        out_shape=jax.ShapeDtypeStruct((q, 1), jnp.int32),
        scratch_shapes=[
            pltpu.VMEM((q, 1), jnp.float32),
            pltpu.VMEM((q, 1), jnp.int32),
        ],
    )(vis, train_db)


def _gather_captions(lang_ids, best):
    n = best.shape[0]
    mesh = plsc.VectorSubcoreMesh(core_axis_name="c", subcore_axis_name="s")

    @functools.partial(
        pl.kernel,
        mesh=mesh,
        out_type=jax.ShapeDtypeStruct((n,), lang_ids.dtype),
        scratch_types=[
            pltpu.VMEM((n,), jnp.int32),
            pltpu.VMEM((n,), lang_ids.dtype),
            pltpu.SemaphoreType.DMA,
        ],
    )
    def k(lang_hbm, idx_hbm, out_hbm, idx_v, vals_v, sem):
        first = (lax.axis_index("c") == 0) & (lax.axis_index("s") == 0)

        @pl.when(first)
        def _():
            pltpu.sync_copy(idx_hbm, idx_v)
            pltpu.async_copy(lang_hbm.at[idx_v], vals_v, sem).wait()
            pltpu.sync_copy(vals_v, out_hbm)

    return k(lang_ids, best)


def kernel(t_feat, train_db, lang_ids):
    vis = t_feat[:, :-4]
    best = _best_index(vis, train_db).reshape(-1)
    return _gather_captions(lang_ids, best)
